# baseline (speedup 1.0000x reference)
import functools

import jax
import jax.numpy as jnp
from jax import lax
from jax.experimental import pallas as pl
from jax.experimental.pallas import tpu as pltpu
from jax.experimental.pallas import tpu_sc as plsc

_NC = 2
_NS = 16
_NW = _NC * _NS
_L = 16


def _sc_gather_pool(user_table, item_table, user_id, song_id,
                    hist_ids, hist_weights):
    B = user_id.shape[0]
    B_, H = hist_ids.shape
    E = user_table.shape[1]
    b_per_w = B // _NW
    CB = 8
    CH = CB * H
    n_chunks = b_per_w // CB

    mesh = plsc.VectorSubcoreMesh(core_axis_name="c", subcore_axis_name="s")

    @functools.partial(
        pl.kernel,
        mesh=mesh,
        compiler_params=pltpu.CompilerParams(use_tc_tiling_on_sc=False,
                                             needs_layout_passes=False),
        out_type=[
            jax.ShapeDtypeStruct((B, E), jnp.float32),
            jax.ShapeDtypeStruct((B, E), jnp.float32),
            jax.ShapeDtypeStruct((B * E,), jnp.float32),
        ],
        scratch_types=[
            pltpu.VMEM((b_per_w,), jnp.int32),
            pltpu.VMEM((b_per_w, E), jnp.float32),
            pltpu.VMEM((CB, H), jnp.int32),
            pltpu.VMEM((CB, H), jnp.int32),
            pltpu.VMEM((CB, H, E), jnp.float32),
            pltpu.VMEM((CB, H, E), jnp.float32),
            pltpu.VMEM((b_per_w, H), jnp.float32),
            pltpu.VMEM((b_per_w * E,), jnp.float32),
            pltpu.SemaphoreType.DMA,
            pltpu.SemaphoreType.DMA,
            pltpu.SemaphoreType.DMA,
        ],
    )
    def gather_kernel(ut_hbm, it_hbm, uid_hbm, sid_hbm, hid_hbm, hw_hbm,
                      u_out, v_out, p_out,
                      idx_b, rows_b, idx_h0, idx_h1, rows_h0, rows_h1,
                      wv, pool_buf, sem_u, sem0, sem1):
        wid = lax.axis_index("s") * _NC + lax.axis_index("c")
        base = wid * b_per_w

        wcopy = pltpu.make_async_copy(
            hw_hbm.at[pl.ds(base, b_per_w)], wv, sem1)
        wcopy.start()

        pltpu.sync_copy(uid_hbm.at[pl.ds(base, b_per_w)], idx_b)
        pltpu.async_copy(ut_hbm.at[idx_b], rows_b, sem_u).wait()
        pltpu.sync_copy(rows_b, u_out.at[pl.ds(base, b_per_w)])
        pltpu.sync_copy(sid_hbm.at[pl.ds(base, b_per_w)], idx_b)
        pltpu.async_copy(it_hbm.at[idx_b], rows_b, sem_u).wait()
        pltpu.sync_copy(rows_b, v_out.at[pl.ds(base, b_per_w)])
        wcopy.wait()

        col = [lax.iota(jnp.int32, _L) + k * _L for k in range(E // _L)]

        def start_gather(c, idx_h, rows_h, sem):
            pltpu.sync_copy(hid_hbm.at[pl.ds(base + c * CB, CB)], idx_h)
            for b in range(CB):
                pltpu.make_async_copy(
                    it_hbm.at[idx_h.at[b]], rows_h.at[b], sem).start()

        def compute_chunk(c, idx_h, rows_h, sem):
            for b in range(CB):
                pltpu.make_async_copy(
                    it_hbm.at[idx_h.at[b]], rows_h.at[b], sem).wait()

            @pl.loop(0, CB)
            def _(b):
                bvec_l = jnp.full((_L,), b, dtype=jnp.int32)
                bvec_w = jnp.full((_L,), c * CB + b, dtype=jnp.int32)

                def jstep(j, acc):
                    jvec = jnp.full((_L,), j, dtype=jnp.int32)
                    wvec = plsc.load_gather(wv, [bvec_w, jvec])
                    return tuple(
                        acc[k] + wvec * plsc.load_gather(
                            rows_h, [bvec_l, jvec, col[k]])
                        for k in range(E // _L))

                acc = lax.fori_loop(
                    0, H, jstep,
                    tuple(jnp.zeros((_L,), jnp.float32)
                          for _ in range(E // _L)))
                pbase = (c * CB + b) * E
                for k in range(E // _L):
                    pool_buf[pl.ds(pbase + k * _L, _L)] = acc[k]

        start_gather(0, idx_h0, rows_h0, sem0)

        @pl.loop(0, n_chunks // 2)
        def _(cc):
            c = cc * 2

            start_gather(c + 1, idx_h1, rows_h1, sem1)
            compute_chunk(c, idx_h0, rows_h0, sem0)

            @pl.when(c + 2 < n_chunks)
            def _():
                start_gather(c + 2, idx_h0, rows_h0, sem0)
            compute_chunk(c + 1, idx_h1, rows_h1, sem1)

        pltpu.sync_copy(pool_buf, p_out.at[pl.ds(base * E, b_per_w * E)])

    return gather_kernel(user_table, item_table, user_id, song_id,
                         hist_ids, hist_weights)


def _tc_mlp(u, v, pooled, hist_weights, W1, b1, W2, b2, W3, b3, W_out, b_out):
    B, E = u.shape
    H = hist_weights.shape[1]
    BS = 512
    grid = (B // BS,)

    def body(u_ref, v_ref, p_ref, w_ref,
             W1_ref, b1_ref, W2_ref, b2_ref, W3_ref, b3_ref,
             Wo_ref, bo_ref, out_ref):
        w = w_ref[...]
        wsum = jnp.sum(w, axis=1, keepdims=True) + 1e-8
        hist = p_ref[...] / wsum
        uu = u_ref[...]
        vv = v_ref[...]
        x = jnp.concatenate([uu, vv, hist], axis=1)
        x = jnp.maximum(jnp.dot(x, W1_ref[...],
                                preferred_element_type=jnp.float32)
                        + b1_ref[...][None, :], 0.0)
        x = jnp.maximum(jnp.dot(x, W2_ref[...],
                                preferred_element_type=jnp.float32)
                        + b2_ref[...][None, :], 0.0)
        x = jnp.maximum(jnp.dot(x, W3_ref[...],
                                preferred_element_type=jnp.float32)
                        + b3_ref[...][None, :], 0.0)
        y = jnp.concatenate([uu * vv, x], axis=1)
        out_ref[...] = (jnp.dot(y, Wo_ref[...],
                                preferred_element_type=jnp.float32)
                        + bo_ref[...][None, :])

    rep = lambda *shape: pl.BlockSpec(shape, lambda i: (0,) * len(shape))
    return pl.pallas_call(
        body,
        grid=grid,
        in_specs=[
            pl.BlockSpec((BS, E), lambda i: (i, 0)),
            pl.BlockSpec((BS, E), lambda i: (i, 0)),
            pl.BlockSpec((BS, E), lambda i: (i, 0)),
            pl.BlockSpec((BS, H), lambda i: (i, 0)),
            rep(*W1.shape), rep(*b1.shape),
            rep(*W2.shape), rep(*b2.shape),
            rep(*W3.shape), rep(*b3.shape),
            rep(*W_out.shape), rep(*b_out.shape),
        ],
        out_specs=pl.BlockSpec((BS, 1), lambda i: (i, 0)),
        out_shape=jax.ShapeDtypeStruct((B, 1), jnp.float32),
    )(u, v, pooled, hist_weights, W1, b1, W2, b2, W3, b3, W_out, b_out)


def kernel(user_id, song_id, hist_ids, hist_weights, user_table, item_table,
           W1, b1, W2, b2, W3, b3, W_out, b_out):
    B, H = hist_ids.shape
    E = user_table.shape[1]
    u, v, pooled_flat = _sc_gather_pool(user_table, item_table,
                                        user_id, song_id,
                                        hist_ids, hist_weights)
    pooled = pooled_flat.reshape(B, E)
    return _tc_mlp(u, v, pooled, hist_weights,
                   W1, b1, W2, b2, W3, b3, W_out, b_out)
